# Initial kernel scaffold; baseline (speedup 1.0000x reference)
import jax
import jax.numpy as jnp
from jax.experimental import pallas as pl


def kernel(x):
    raise NotImplementedError("write your pallas kernel here")


# speedup vs baseline: 36.4042x; 36.4042x over previous
import jax
import jax.numpy as jnp
from jax.experimental import pallas as pl
from jax.experimental.pallas import tpu as pltpu


def _kwta_row_kernel(k, x_hbm, out_hbm, buf, key_buf, in_sem, out_sem):
    b = pl.program_id(0)
    cp_in = pltpu.make_async_copy(x_hbm.at[b], buf, in_sem)
    cp_in.start()
    cp_in.wait()

    xv = buf[...]
    ibits = pltpu.bitcast(xv, jnp.int32)
    skey = ibits ^ jnp.bitwise_and(
        jnp.right_shift(ibits, 31), jnp.int32(0x7FFFFFFF)
    )
    key_buf[...] = skey

    def step(i, p):
        cand = p | (jnp.int32(1) << (31 - i))
        st = cand ^ jnp.int32(-2147483648)
        cnt = jnp.sum((key_buf[...] >= st).astype(jnp.int32))
        return jnp.where(cnt >= k, cand, p)

    p_final = jax.lax.fori_loop(0, 32, step, jnp.int32(0), unroll=False)
    kth_key = p_final ^ jnp.int32(-2147483648)

    buf[...] = jnp.where(key_buf[...] >= kth_key, xv, jnp.float32(0.0))

    cp_out = pltpu.make_async_copy(buf, out_hbm.at[b], out_sem)
    cp_out.start()
    cp_out.wait()


def _kwta_2d(x2d, k, interpret=False):
    b, n = x2d.shape
    assert n % 1024 == 0, n
    rows = n // 128
    x3 = x2d.reshape(b, rows, 128)
    import functools

    out = pl.pallas_call(
        functools.partial(_kwta_row_kernel, k),
        grid=(b,),
        in_specs=[pl.BlockSpec(memory_space=pl.ANY)],
        out_specs=pl.BlockSpec(memory_space=pl.ANY),
        out_shape=jax.ShapeDtypeStruct((b, rows, 128), jnp.float32),
        scratch_shapes=[
            pltpu.VMEM((rows, 128), jnp.float32),
            pltpu.VMEM((rows, 128), jnp.int32),
            pltpu.SemaphoreType.DMA,
            pltpu.SemaphoreType.DMA,
        ],
        interpret=interpret,
    )(x3)
    return out.reshape(b, n)


def kernel(x):
    b = x.shape[0]
    size = x.shape[1] * x.shape[2] * x.shape[3]
    k = int(0.1 * size)
    out = _kwta_2d(x.reshape(b, size), k)
    return out.reshape(x.shape)


# speedup vs baseline: 41.7763x; 1.1476x over previous
import jax
import jax.numpy as jnp
from jax.experimental import pallas as pl
from jax.experimental.pallas import tpu as pltpu


def _kwta_row_kernel(k, x_hbm, out_hbm, buf, key_buf, in_sem, out_sem):
    b = pl.program_id(0)
    cp_in = pltpu.make_async_copy(x_hbm.at[b], buf, in_sem)
    cp_in.start()
    cp_in.wait()

    xv = buf[...]
    ibits = pltpu.bitcast(xv, jnp.int32)
    skey = ibits ^ jnp.bitwise_and(
        jnp.right_shift(ibits, 31), jnp.int32(0x7FFFFFFF)
    )
    key_buf[...] = skey

    rows = key_buf.shape[0]
    acc_rows = 64

    def step(i, p):
        cand = p | (jnp.int32(1) << (31 - i))
        st = cand ^ jnp.int32(-2147483648)

        def chunk(j, acc):
            blk = key_buf[pl.ds(j * acc_rows, acc_rows), :]
            return acc + (blk >= st).astype(jnp.int32)

        acc = jax.lax.fori_loop(
            0,
            rows // acc_rows,
            chunk,
            jnp.zeros((acc_rows, 128), jnp.int32),
            unroll=False,
        )
        cnt = jnp.sum(acc)
        return jnp.where(cnt >= k, cand, p)

    p_final = jax.lax.fori_loop(0, 32, step, jnp.int32(0), unroll=False)
    kth_key = p_final ^ jnp.int32(-2147483648)

    buf[...] = jnp.where(key_buf[...] >= kth_key, xv, jnp.float32(0.0))

    cp_out = pltpu.make_async_copy(buf, out_hbm.at[b], out_sem)
    cp_out.start()
    cp_out.wait()


def _kwta_2d(x2d, k, interpret=False):
    b, n = x2d.shape
    assert n % 1024 == 0, n
    rows = n // 128
    x3 = x2d.reshape(b, rows, 128)
    import functools

    out = pl.pallas_call(
        functools.partial(_kwta_row_kernel, k),
        grid=(b,),
        in_specs=[pl.BlockSpec(memory_space=pl.ANY)],
        out_specs=pl.BlockSpec(memory_space=pl.ANY),
        out_shape=jax.ShapeDtypeStruct((b, rows, 128), jnp.float32),
        scratch_shapes=[
            pltpu.VMEM((rows, 128), jnp.float32),
            pltpu.VMEM((rows, 128), jnp.int32),
            pltpu.SemaphoreType.DMA,
            pltpu.SemaphoreType.DMA,
        ],
        interpret=interpret,
    )(x3)
    return out.reshape(b, n)


def kernel(x):
    b = x.shape[0]
    size = x.shape[1] * x.shape[2] * x.shape[3]
    k = int(0.1 * size)
    out = _kwta_2d(x.reshape(b, size), k)
    return out.reshape(x.shape)


# speedup vs baseline: 56.4493x; 1.3512x over previous
import jax
import jax.numpy as jnp
from jax.experimental import pallas as pl
from jax.experimental.pallas import tpu as pltpu


def _kwta_row_kernel(k, x_hbm, out_hbm, buf, key_buf, in_sem, out_sem):
    b = pl.program_id(0)
    cp_in = pltpu.make_async_copy(x_hbm.at[b], buf, in_sem)
    cp_in.start()
    cp_in.wait()

    xv = buf[...]
    ibits = pltpu.bitcast(xv, jnp.int32)
    skey = ibits ^ jnp.bitwise_and(
        jnp.right_shift(ibits, 31), jnp.int32(0x7FFFFFFF)
    )
    key_buf[...] = skey

    rows = key_buf.shape[0]
    acc_rows = 64

    def step(i, p):
        cand = p | (jnp.int32(1) << (31 - i))
        st = cand ^ jnp.int32(-2147483648)

        def chunk(j, acc):
            blk = key_buf[pl.ds(j * acc_rows, acc_rows), :]
            return acc + (blk >= st).astype(jnp.int32)

        acc = jax.lax.fori_loop(
            0,
            rows // acc_rows,
            chunk,
            jnp.zeros((acc_rows, 128), jnp.int32),
            unroll=4,
        )
        cnt = jnp.sum(acc)
        return jnp.where(cnt >= k, cand, p)

    p_final = jax.lax.fori_loop(0, 32, step, jnp.int32(0), unroll=False)
    kth_key = p_final ^ jnp.int32(-2147483648)

    buf[...] = jnp.where(key_buf[...] >= kth_key, xv, jnp.float32(0.0))

    cp_out = pltpu.make_async_copy(buf, out_hbm.at[b], out_sem)
    cp_out.start()
    cp_out.wait()


def _kwta_2d(x2d, k, interpret=False):
    b, n = x2d.shape
    assert n % 1024 == 0, n
    rows = n // 128
    x3 = x2d.reshape(b, rows, 128)
    import functools

    out = pl.pallas_call(
        functools.partial(_kwta_row_kernel, k),
        grid=(b,),
        in_specs=[pl.BlockSpec(memory_space=pl.ANY)],
        out_specs=pl.BlockSpec(memory_space=pl.ANY),
        out_shape=jax.ShapeDtypeStruct((b, rows, 128), jnp.float32),
        scratch_shapes=[
            pltpu.VMEM((rows, 128), jnp.float32),
            pltpu.VMEM((rows, 128), jnp.int32),
            pltpu.SemaphoreType.DMA,
            pltpu.SemaphoreType.DMA,
        ],
        interpret=interpret,
    )(x3)
    return out.reshape(b, n)


def kernel(x):
    b = x.shape[0]
    size = x.shape[1] * x.shape[2] * x.shape[3]
    k = int(0.1 * size)
    out = _kwta_2d(x.reshape(b, size), k)
    return out.reshape(x.shape)


# speedup vs baseline: 59.9434x; 1.0619x over previous
import jax
import jax.numpy as jnp
from jax.experimental import pallas as pl
from jax.experimental.pallas import tpu as pltpu


def _kwta_row_kernel(k, x_hbm, out_hbm, buf, key_buf, in_sem, out_sem):
    b = pl.program_id(0)
    cp_in = pltpu.make_async_copy(x_hbm.at[b], buf, in_sem)
    cp_in.start()
    cp_in.wait()

    xv = buf[...]
    ibits = pltpu.bitcast(xv, jnp.int32)
    skey = ibits ^ jnp.bitwise_and(
        jnp.right_shift(ibits, 31), jnp.int32(0x7FFFFFFF)
    )
    key_buf[...] = skey

    rows = key_buf.shape[0]
    acc_rows = 64

    def step(i, p):
        cand = p | (jnp.int32(1) << (31 - i))
        st = cand ^ jnp.int32(-2147483648)

        def chunk(j, acc):
            blk = key_buf[pl.ds(j * acc_rows, acc_rows), :]
            return acc + (blk >= st).astype(jnp.int32)

        acc = jax.lax.fori_loop(
            0,
            rows // acc_rows,
            chunk,
            jnp.zeros((acc_rows, 128), jnp.int32),
            unroll=8,
        )
        cnt = jnp.sum(acc)
        return jnp.where(cnt >= k, cand, p)

    p_final = jax.lax.fori_loop(0, 32, step, jnp.int32(0), unroll=False)
    kth_key = p_final ^ jnp.int32(-2147483648)

    buf[...] = jnp.where(key_buf[...] >= kth_key, xv, jnp.float32(0.0))

    cp_out = pltpu.make_async_copy(buf, out_hbm.at[b], out_sem)
    cp_out.start()
    cp_out.wait()


def _kwta_2d(x2d, k, interpret=False):
    b, n = x2d.shape
    assert n % 1024 == 0, n
    rows = n // 128
    x3 = x2d.reshape(b, rows, 128)
    import functools

    out = pl.pallas_call(
        functools.partial(_kwta_row_kernel, k),
        grid=(b,),
        in_specs=[pl.BlockSpec(memory_space=pl.ANY)],
        out_specs=pl.BlockSpec(memory_space=pl.ANY),
        out_shape=jax.ShapeDtypeStruct((b, rows, 128), jnp.float32),
        scratch_shapes=[
            pltpu.VMEM((rows, 128), jnp.float32),
            pltpu.VMEM((rows, 128), jnp.int32),
            pltpu.SemaphoreType.DMA,
            pltpu.SemaphoreType.DMA,
        ],
        interpret=interpret,
    )(x3)
    return out.reshape(b, n)


def kernel(x):
    b = x.shape[0]
    size = x.shape[1] * x.shape[2] * x.shape[3]
    k = int(0.1 * size)
    out = _kwta_2d(x.reshape(b, size), k)
    return out.reshape(x.shape)
